# Initial kernel scaffold; baseline (speedup 1.0000x reference)
import jax
import jax.numpy as jnp
from jax.experimental import pallas as pl


def kernel(x, conv1_w, l1b0_w1, l1b0_w2, l2b0_w1, l2b0_w2, l2b0_wds, l3b0_w1, l3b0_w2, l3b0_wds, l4b0_w1, l4b0_w2, l4b0_wds):
    raise NotImplementedError("write your pallas kernel here")


# speedup vs baseline: 1.5678x; 1.5678x over previous
import jax
import jax.numpy as jnp
from jax.experimental import pallas as pl
from jax.experimental.pallas import tpu as pltpu


def _rup(x, m):
    return ((x + m - 1) // m) * m


def _mtile(m):
    if m >= 4096:
        return 512
    return _rup((m + 3) // 4, 16)


def _etile(mp):
    if mp % 512 == 0:
        return 512
    q = mp // 4
    if mp % 4 == 0 and q % 16 == 0:
        return q
    return mp


def _gemm_bnstat_kernel(a_ref, w_ref, o_ref, st_ref):
    acc = jnp.dot(a_ref[...], w_ref[...], preferred_element_type=jnp.float32)
    o_ref[...] = acc.astype(jnp.bfloat16)
    s = jnp.sum(acc, axis=0, keepdims=True)
    q = jnp.sum(acc * acc, axis=0, keepdims=True)
    st_ref[...] = jnp.concatenate([jnp.broadcast_to(s, (4, s.shape[1])),
                                   jnp.broadcast_to(q, (4, q.shape[1]))], axis=0)


def _affine_relu_kernel(x_ref, p_ref, o_ref):
    p = p_ref[...]
    y = x_ref[...].astype(jnp.float32) * p[0:1] + p[1:2]
    o_ref[...] = jnp.maximum(y, 0.0).astype(jnp.bfloat16)


def _affine_res_relu_kernel(x_ref, p_ref, r_ref, o_ref):
    p = p_ref[...]
    y = (x_ref[...].astype(jnp.float32) * p[0:1] + p[1:2]
         + r_ref[...].astype(jnp.float32))
    o_ref[...] = jnp.maximum(y, 0.0).astype(jnp.bfloat16)


def _affine2_res_relu_kernel(x_ref, p_ref, r_ref, o_ref):
    p = p_ref[...]
    y = (x_ref[...].astype(jnp.float32) * p[0:1] + p[1:2]
         + r_ref[...].astype(jnp.float32) * p[2:3] + p[3:4])
    o_ref[...] = jnp.maximum(y, 0.0).astype(jnp.bfloat16)


def _stem_pool_kernel(p00_ref, p01_ref, p10_ref, p11_ref, p_ref, o_ref):
    a, b, c, d = p00_ref[0], p01_ref[0], p10_ref[0], p11_ref[0]
    ho, wo = a.shape[0] - 1, a.shape[1] - 1
    taps = (a[0:ho, 0:wo], a[0:ho, 1:wo + 1], a[1:ho + 1, 0:wo],
            a[1:ho + 1, 1:wo + 1],
            b[0:ho, 0:wo], b[1:ho + 1, 0:wo],
            c[0:ho, 0:wo], c[0:ho, 1:wo + 1],
            d[0:ho, 0:wo])
    best = taps[0]
    for t in taps[1:]:
        best = jnp.maximum(best, t)
    p = p_ref[...]
    y = best.astype(jnp.float32) * p[0:1] + p[1:2]
    o_ref[0] = jnp.maximum(y, 0.0).astype(jnp.bfloat16)


def _conv_gemm(patches, w, tm):
    mp, kp = patches.shape
    n = w.shape[1]
    g = mp // tm
    return pl.pallas_call(
        _gemm_bnstat_kernel,
        out_shape=(jax.ShapeDtypeStruct((mp, n), jnp.bfloat16),
                   jax.ShapeDtypeStruct((g * 8, n), jnp.float32)),
        grid=(g,),
        in_specs=[pl.BlockSpec((tm, kp), lambda i: (i, 0)),
                  pl.BlockSpec((kp, n), lambda i: (0, 0))],
        out_specs=(pl.BlockSpec((tm, n), lambda i: (i, 0)),
                   pl.BlockSpec((8, n), lambda i: (i, 0))),
        compiler_params=pltpu.CompilerParams(
            dimension_semantics=("parallel",),
            vmem_limit_bytes=64 * 1024 * 1024),
    )(patches, w)


def _bn_affine(stats, m, eps=1e-5):
    n = stats.shape[1]
    st = stats.reshape(-1, 2, 4, n)
    mean = jnp.sum(st[:, 0, 0, :], axis=0, keepdims=True) / float(m)
    ex2 = jnp.sum(st[:, 1, 0, :], axis=0, keepdims=True) / float(m)
    var = jnp.maximum(ex2 - mean * mean, 0.0)
    sc = jax.lax.rsqrt(var + eps)
    return jnp.concatenate([sc, -mean * sc], axis=0)


def _bn_apply(x2d, p, res=None, res_p=None):
    mp, c = x2d.shape
    tm = _etile(mp)
    big = pl.BlockSpec((tm, c), lambda i: (i, 0))
    if res is None:
        kfn = _affine_relu_kernel
        args = (x2d, p)
        specs = [big, pl.BlockSpec((2, c), lambda i: (0, 0))]
    elif res_p is None:
        kfn = _affine_res_relu_kernel
        args = (x2d, p, res)
        specs = [big, pl.BlockSpec((2, c), lambda i: (0, 0)), big]
    else:
        kfn = _affine2_res_relu_kernel
        args = (x2d, jnp.concatenate([p, res_p], axis=0), res)
        specs = [big, pl.BlockSpec((4, c), lambda i: (0, 0)), big]
    return pl.pallas_call(
        kfn,
        out_shape=jax.ShapeDtypeStruct((mp, c), jnp.bfloat16),
        grid=(mp // tm,),
        in_specs=specs,
        out_specs=big,
        compiler_params=pltpu.CompilerParams(
            dimension_semantics=("parallel",)),
    )(*args)


def _im2col(x, kh, kw, stride, pad, tm, kp):
    nb, h, w, c = x.shape
    ho = (h + 2 * pad - kh) // stride + 1
    wo = (w + 2 * pad - kw) // stride + 1
    if pad:
        x = jnp.pad(x, ((0, 0), (pad, pad), (pad, pad), (0, 0)))
    taps = [x[:, i:i + stride * ho:stride, j:j + stride * wo:stride, :]
            for i in range(kh) for j in range(kw)]
    m = nb * ho * wo
    k = kh * kw * c
    pm = jnp.concatenate(taps, axis=-1).reshape(m, k)
    mp = _rup(m, tm)
    pm = jnp.pad(pm, ((0, mp - m), (0, kp - k)))
    return pm, m, (nb, ho, wo)


def _conv_bn(x, w, kh, kw, stride, pad):
    nb, h, wd, _ = x.shape
    ho = (h + 2 * pad - kh) // stride + 1
    wo = (wd + 2 * pad - kw) // stride + 1
    tm = _mtile(nb * ho * wo)
    pm, m, _ = _im2col(x, kh, kw, stride, pad, tm, w.shape[0])
    out, stats = _conv_gemm(pm, w, tm)
    return out, _bn_affine(stats, m), m, (nb, ho, wo, w.shape[1])


def _stem_pool(raw_nhwc, p):
    nb, h, w, c = raw_nhwc.shape
    ho, wo = h // 2, w // 2
    xp = jnp.pad(raw_nhwc, ((0, 0), (1, 1), (1, 1), (0, 0)),
                 constant_values=-jnp.inf)
    hp = ho + 1
    phases = [xp[:, i::2, j::2, :] for i in range(2) for j in range(2)]
    ph_spec = pl.BlockSpec((1, hp, hp, c), lambda i: (i, 0, 0, 0))
    return pl.pallas_call(
        _stem_pool_kernel,
        out_shape=jax.ShapeDtypeStruct((nb, ho, wo, c), jnp.bfloat16),
        grid=(nb,),
        in_specs=[ph_spec, ph_spec, ph_spec, ph_spec,
                  pl.BlockSpec((2, c), lambda i: (0, 0))],
        out_specs=pl.BlockSpec((1, ho, wo, c), lambda i: (i, 0, 0, 0)),
        compiler_params=pltpu.CompilerParams(
            dimension_semantics=("parallel",)),
    )(*phases, p)


def _block(x, w1, w2, wds, stride):
    out1, p1, m1, shp1 = _conv_bn(x, w1, 3, 3, stride, 1)
    a1 = _bn_apply(out1, p1)[:m1].reshape(shp1)
    out2, p2, m2, shp2 = _conv_bn(a1, w2, 3, 3, 1, 1)
    if wds is None:
        res = x.reshape(-1, x.shape[-1])
        res = jnp.pad(res, ((0, out2.shape[0] - m2), (0, 0)))
        y = _bn_apply(out2, p2, res=res)
    else:
        d, pd, _, _ = _conv_bn(x, wds, 1, 1, stride, 0)
        y = _bn_apply(out2, p2, res=d, res_p=pd)
    return y[:m2].reshape(shp2)


def kernel(x, conv1_w, l1b0_w1, l1b0_w2, l2b0_w1, l2b0_w2, l2b0_wds,
           l3b0_w1, l3b0_w2, l3b0_wds, l4b0_w1, l4b0_w2, l4b0_wds):
    xh = jnp.transpose(x, (0, 2, 3, 1)).astype(jnp.bfloat16)
    raw0, p0, m0, shp0 = _conv_bn(xh, conv1_w, 7, 7, 2, 3)
    out0 = _bn_apply(raw0, p0)[:m0].reshape(shp0)
    pooled = _stem_pool(raw0[:m0].reshape(shp0), p0)
    f1 = _block(pooled, l1b0_w1, l1b0_w2, None, 1)
    f2 = _block(f1, l2b0_w1, l2b0_w2, l2b0_wds, 2)
    f3 = _block(f2, l3b0_w1, l3b0_w2, l3b0_wds, 2)
    f4 = _block(f3, l4b0_w1, l4b0_w2, l4b0_wds, 2)
    return [jnp.transpose(f, (0, 3, 1, 2)).astype(jnp.float32)
            for f in (out0, f1, f2, f3, f4)]
